# Initial kernel scaffold; baseline (speedup 1.0000x reference)
import jax
import jax.numpy as jnp
from jax.experimental import pallas as pl


def kernel(lngs, lats, states, dist_gap, lens, W_state, W_pc, b_pc, W_conv, b_conv, W_ih_l0, W_hh_l0, b_ih_l0, b_hh_l0, W_ih_l1, W_hh_l1, b_ih_l1, b_hh_l1):
    raise NotImplementedError("write your pallas kernel here")


# speedup vs baseline: 4.5762x; 4.5762x over previous
import functools

import jax
import jax.numpy as jnp
from jax.experimental import pallas as pl
from jax.experimental.pallas import tpu as pltpu

B, T = 16, 2048
KSZ, NF, H = 3, 32, 128
DIST_MEAN, DIST_STD = 0.5, 0.29
S = T - KSZ + 1
TB = 186
NBLK = S // TB
TBP = TB + KSZ - 1
G4 = 4 * H

_HI = jax.lax.Precision.HIGHEST


def _dot(a, b):
    return jax.lax.dot_general(a, b, (((1,), (0,)), ((), ())),
                               precision=_HI,
                               preferred_element_type=jnp.float32)


def _body(geo_ref, wst_ref, wpcT_ref, bpc_ref, convW_ref, bconv_ref,
          wih0aT_ref, wd_ref, bih0_ref, bhh0_ref, whh0T_ref,
          w1T_ref, bih1_ref, bhh1_ref, lens_ref,
          out_ref, xg_ref, carry_ref):
    i = pl.program_id(0)
    t0 = i * TB

    geoW = geo_ref[pl.ds(t0 * B, TBP * B), :]

    a0 = wst_ref[0, 0]
    a1 = wst_ref[0, 1]
    d0 = wst_ref[1, 0] - a0
    d1 = wst_ref[1, 1] - a1
    wpcT = wpcT_ref[...]
    wpc_eff = jnp.concatenate(
        [wpcT[0:2], d0 * wpcT[2:3] + d1 * wpcT[3:4],
         jnp.zeros((1, 16), jnp.float32)], axis=0)
    bpc_eff = bpc_ref[...] + a0 * wpcT[2:3] + a1 * wpcT[3:4]

    proj = jnp.tanh(_dot(geoW, wpc_eff) + bpc_eff)

    acc = jnp.zeros((TB * B, NF), jnp.float32)
    for k in range(KSZ):
        acc = acc + _dot(proj[k * B:k * B + TB * B], convW_ref[k])
    pre = acc + bconv_ref[...]
    conv = jnp.where(pre > 0, pre, jnp.exp(jnp.minimum(pre, 0.0)) - 1.0)

    wd = wd_ref[...]
    wdm = jnp.concatenate(
        [jnp.zeros((3, G4), jnp.float32), wd / DIST_STD], axis=0)
    dgd = geoW[(KSZ - 1) * B:] - geoW[:TB * B]

    bias0 = bih0_ref[...] + bhh0_ref[...] - (DIST_MEAN / DIST_STD) * wd
    xg_ref[...] = _dot(conv, wih0aT_ref[...]) + _dot(dgd, wdm) + bias0

    @pl.when(i == 0)
    def _init():
        carry_ref[...] = jnp.zeros((4, B, H), jnp.float32)

    whh0T = whh0T_ref[...]
    w1T = w1T_ref[...]
    bias1 = bih1_ref[...] + bhh1_ref[...]

    def step(t, carry):
        h0, c0, h1, c1 = carry
        g0 = xg_ref[pl.ds(t * B, B), :] + _dot(h0, whh0T)
        ig = jax.nn.sigmoid(g0[:, 0:H])
        fg = jax.nn.sigmoid(g0[:, H:2 * H])
        gg = jnp.tanh(g0[:, 2 * H:3 * H])
        og = jax.nn.sigmoid(g0[:, 3 * H:])
        c0 = fg * c0 + ig * gg
        h0 = og * jnp.tanh(c0)

        g1 = _dot(jnp.concatenate([h0, h1], axis=1), w1T) + bias1
        i1 = jax.nn.sigmoid(g1[:, 0:H])
        f1 = jax.nn.sigmoid(g1[:, H:2 * H])
        gt1 = jnp.tanh(g1[:, 2 * H:3 * H])
        o1 = jax.nn.sigmoid(g1[:, 3 * H:])
        c1 = f1 * c1 + i1 * gt1
        h1 = o1 * jnp.tanh(c1)

        out_ref[t] = h1
        return h0, c0, h1, c1

    carry = (carry_ref[0], carry_ref[1], carry_ref[2], carry_ref[3])
    h0, c0, h1, c1 = jax.lax.fori_loop(0, TB, step, carry)
    carry_ref[0] = h0
    carry_ref[1] = c0
    carry_ref[2] = h1
    carry_ref[3] = c1

    lensc = lens_ref[...] - (KSZ - 1)
    tids = t0 + jax.lax.broadcasted_iota(jnp.int32, (TB, B, 1), 0)
    mask = (tids < lensc[None, :, :]).astype(jnp.float32)
    out_ref[...] = out_ref[...] * mask


@functools.partial(jax.jit, static_argnums=())
def kernel(lngs, lats, states, dist_gap, lens, W_state, W_pc, b_pc, W_conv,
           b_conv, W_ih_l0, W_hh_l0, b_ih_l0, b_hh_l0, W_ih_l1, W_hh_l1,
           b_ih_l1, b_hh_l1):
    geo = jnp.stack(
        [lngs, lats, states.astype(jnp.float32), dist_gap], axis=-1)
    geo = jnp.transpose(geo, (1, 0, 2)).reshape(T * B, 4)

    wpcT = W_pc.T
    convW = jnp.transpose(W_conv, (2, 1, 0))
    wih0aT = W_ih_l0[:, :NF].T
    wd = W_ih_l0[:, NF][None, :]
    whh0T = W_hh_l0.T
    w1T = jnp.concatenate([W_ih_l1, W_hh_l1], axis=1).T

    full = lambda shp: pl.BlockSpec(shp, lambda i: tuple(0 for _ in shp))
    out = pl.pallas_call(
        _body,
        grid=(NBLK,),
        in_specs=[
            full((T * B, 4)),
            full((2, 2)),
            full((4, 16)),
            full((1, 16)),
            full((KSZ, 16, NF)),
            full((1, NF)),
            full((NF, G4)),
            full((1, G4)),
            full((1, G4)),
            full((1, G4)),
            full((H, G4)),
            full((2 * H, G4)),
            full((1, G4)),
            full((1, G4)),
            full((B, 1)),
        ],
        out_specs=pl.BlockSpec((TB, B, H), lambda i: (i, 0, 0)),
        out_shape=jax.ShapeDtypeStruct((S, B, H), jnp.float32),
        scratch_shapes=[
            pltpu.VMEM((TB * B, G4), jnp.float32),
            pltpu.VMEM((4, B, H), jnp.float32),
        ],
    )(geo, W_state, wpcT, b_pc[None, :], convW, b_conv[None, :],
      wih0aT, wd, b_ih_l0[None, :], b_hh_l0[None, :], whh0T,
      w1T, b_ih_l1[None, :], b_hh_l1[None, :], lens[:, None])

    h_local = jnp.transpose(out, (1, 0, 2))
    return h_local, lens - (KSZ - 1)


# speedup vs baseline: 10.1501x; 2.2180x over previous
import functools

import jax
import jax.numpy as jnp
from jax.experimental import pallas as pl
from jax.experimental.pallas import tpu as pltpu

B, T = 16, 2048
KSZ, NF, H = 3, 32, 128
DIST_MEAN, DIST_STD = 0.5, 0.29
S = T - KSZ + 1
TB = 186
NBLK = S // TB
TBP = TB + KSZ - 1
G4 = 4 * H

_HI = jax.lax.Precision.DEFAULT


def _dot(a, b):
    return jax.lax.dot_general(a, b, (((1,), (0,)), ((), ())),
                               precision=_HI,
                               preferred_element_type=jnp.float32)


def _body(geo_ref, wst_ref, wpcT_ref, bpc_ref, convW_ref, bconv_ref,
          wih0aT_ref, wd_ref, bih0_ref, bhh0_ref, whh0T_ref,
          w1T_ref, bih1_ref, bhh1_ref, lens_ref,
          out_ref, xg_ref, carry_ref):
    i = pl.program_id(0)
    t0 = i * TB

    geoW = geo_ref[pl.ds(t0 * B, TBP * B), :]

    a0 = wst_ref[0, 0]
    a1 = wst_ref[0, 1]
    d0 = wst_ref[1, 0] - a0
    d1 = wst_ref[1, 1] - a1
    wpcT = wpcT_ref[...]
    wpc_eff = jnp.concatenate(
        [wpcT[0:2], d0 * wpcT[2:3] + d1 * wpcT[3:4],
         jnp.zeros((1, 16), jnp.float32)], axis=0)
    bpc_eff = bpc_ref[...] + a0 * wpcT[2:3] + a1 * wpcT[3:4]

    proj = jnp.tanh(_dot(geoW, wpc_eff) + bpc_eff)

    acc = jnp.zeros((TB * B, NF), jnp.float32)
    for k in range(KSZ):
        acc = acc + _dot(proj[k * B:k * B + TB * B], convW_ref[k])
    pre = acc + bconv_ref[...]
    conv = jnp.where(pre > 0, pre, jnp.exp(jnp.minimum(pre, 0.0)) - 1.0)

    wd = wd_ref[...]
    wdm = jnp.concatenate(
        [jnp.zeros((3, G4), jnp.float32), wd / DIST_STD], axis=0)
    dgd = geoW[(KSZ - 1) * B:] - geoW[:TB * B]

    bias0 = bih0_ref[...] + bhh0_ref[...] - (DIST_MEAN / DIST_STD) * wd
    xg_ref[...] = _dot(conv, wih0aT_ref[...]) + _dot(dgd, wdm) + bias0

    @pl.when(i == 0)
    def _init():
        carry_ref[...] = jnp.zeros((4, B, H), jnp.float32)

    whh0T = whh0T_ref[...]
    w1T = w1T_ref[...]
    bias1 = bih1_ref[...] + bhh1_ref[...]

    def step(t, carry):
        h0, c0, h1, c1 = carry
        g0 = xg_ref[pl.ds(t * B, B), :] + _dot(h0, whh0T)
        ig = jax.nn.sigmoid(g0[:, 0:H])
        fg = jax.nn.sigmoid(g0[:, H:2 * H])
        gg = jnp.tanh(g0[:, 2 * H:3 * H])
        og = jax.nn.sigmoid(g0[:, 3 * H:])
        c0 = fg * c0 + ig * gg
        h0 = og * jnp.tanh(c0)

        g1 = _dot(jnp.concatenate([h0, h1], axis=1), w1T) + bias1
        i1 = jax.nn.sigmoid(g1[:, 0:H])
        f1 = jax.nn.sigmoid(g1[:, H:2 * H])
        gt1 = jnp.tanh(g1[:, 2 * H:3 * H])
        o1 = jax.nn.sigmoid(g1[:, 3 * H:])
        c1 = f1 * c1 + i1 * gt1
        h1 = o1 * jnp.tanh(c1)

        out_ref[t] = h1
        return h0, c0, h1, c1

    carry = (carry_ref[0], carry_ref[1], carry_ref[2], carry_ref[3])
    h0, c0, h1, c1 = jax.lax.fori_loop(0, TB, step, carry)
    carry_ref[0] = h0
    carry_ref[1] = c0
    carry_ref[2] = h1
    carry_ref[3] = c1

    lensc = lens_ref[...] - (KSZ - 1)
    tids = t0 + jax.lax.broadcasted_iota(jnp.int32, (TB, B, 1), 0)
    mask = (tids < lensc[None, :, :]).astype(jnp.float32)
    out_ref[...] = out_ref[...] * mask


@functools.partial(jax.jit, static_argnums=())
def kernel(lngs, lats, states, dist_gap, lens, W_state, W_pc, b_pc, W_conv,
           b_conv, W_ih_l0, W_hh_l0, b_ih_l0, b_hh_l0, W_ih_l1, W_hh_l1,
           b_ih_l1, b_hh_l1):
    geo = jnp.stack(
        [lngs, lats, states.astype(jnp.float32), dist_gap], axis=-1)
    geo = jnp.transpose(geo, (1, 0, 2)).reshape(T * B, 4)

    wpcT = W_pc.T
    convW = jnp.transpose(W_conv, (2, 1, 0))
    wih0aT = W_ih_l0[:, :NF].T
    wd = W_ih_l0[:, NF][None, :]
    whh0T = W_hh_l0.T
    w1T = jnp.concatenate([W_ih_l1, W_hh_l1], axis=1).T

    full = lambda shp: pl.BlockSpec(shp, lambda i: tuple(0 for _ in shp))
    out = pl.pallas_call(
        _body,
        grid=(NBLK,),
        in_specs=[
            full((T * B, 4)),
            full((2, 2)),
            full((4, 16)),
            full((1, 16)),
            full((KSZ, 16, NF)),
            full((1, NF)),
            full((NF, G4)),
            full((1, G4)),
            full((1, G4)),
            full((1, G4)),
            full((H, G4)),
            full((2 * H, G4)),
            full((1, G4)),
            full((1, G4)),
            full((B, 1)),
        ],
        out_specs=pl.BlockSpec((TB, B, H), lambda i: (i, 0, 0)),
        out_shape=jax.ShapeDtypeStruct((S, B, H), jnp.float32),
        scratch_shapes=[
            pltpu.VMEM((TB * B, G4), jnp.float32),
            pltpu.VMEM((4, B, H), jnp.float32),
        ],
    )(geo, W_state, wpcT, b_pc[None, :], convW, b_conv[None, :],
      wih0aT, wd, b_ih_l0[None, :], b_hh_l0[None, :], whh0T,
      w1T, b_ih_l1[None, :], b_hh_l1[None, :], lens[:, None])

    h_local = jnp.transpose(out, (1, 0, 2))
    return h_local, lens - (KSZ - 1)
